# Initial kernel scaffold; baseline (speedup 1.0000x reference)
import jax
import jax.numpy as jnp
from jax.experimental import pallas as pl


def kernel(edge_src, edge_dst, edge_scalar_attr, edge_attr, edge_weight_cutoff, node_f, Wk1, Wk2, Wk3, Wv1, Wv2, Wv3, Wdot, Wlin):
    raise NotImplementedError("write your pallas kernel here")


# speedup vs baseline: 1.6127x; 1.6127x over previous
import functools
import math

import jax
import jax.numpy as jnp
from jax import lax
from jax.experimental import pallas as pl
from jax.experimental.pallas import tpu as pltpu
from jax.experimental.pallas import tpu_sc as plsc

N_NODES = 10000
N_EDGES = 320000
D = 128
H = 4
DH = D // H
N_RADIAL = 16
HIDDEN = 128

EDGE_BLOCK = 1000
NODE_BLOCK = 1000

NC = 2
NS = 16
NW = NC * NS
LANES = 16


def _head_expand_mat(dtype=jnp.float32):
  col = lax.broadcasted_iota(jnp.int32, (H, D), 1) // DH
  row = lax.broadcasted_iota(jnp.int32, (H, D), 0)
  return (col == row).astype(dtype)


def _edge_body(esa_ref, xs_ref, xd_ref, ea_ref, cut_ref,
               wk1_ref, wk2_ref, wk3_ref, wv1_ref, wv2_ref, wv3_ref,
               wdot_ref, exp_ref, u_ref):
  f32 = jnp.float32
  s_in = 1.0 / math.sqrt(N_RADIAL)
  s_h = 1.0 / math.sqrt(HIDDEN)

  esa = esa_ref[...]
  hk = jax.nn.gelu(jnp.dot(esa, wk1_ref[...], preferred_element_type=f32) * s_in)
  hk = jax.nn.gelu(jnp.dot(hk, wk2_ref[...], preferred_element_type=f32) * s_h)
  wk = jnp.dot(hk, wk3_ref[...], preferred_element_type=f32) * s_h
  hv = jax.nn.gelu(jnp.dot(esa, wv1_ref[...], preferred_element_type=f32) * s_in)
  hv = jax.nn.gelu(jnp.dot(hv, wv2_ref[...], preferred_element_type=f32) * s_h)
  wv = jnp.dot(hv, wv3_ref[...], preferred_element_type=f32) * s_h

  xs = xs_ref[...]
  ea = ea_ref[...]
  ek = wk * xs * ea
  m = jnp.dot(ek, wdot_ref[...], preferred_element_type=f32)
  xd = xd_ref[...]
  dots = []
  for w in range(H):
    dots.append(jnp.sum(m[:, w * D:(w + 1) * D] * xd, axis=1, keepdims=True))
  dot = jnp.concatenate(dots, axis=1) * (1.0 / D)

  cut = cut_ref[...]
  edot2 = jnp.exp(0.5 * dot)
  exp_ref[...] = cut * edot2 * edot2
  sexp = jnp.sqrt(cut) * edot2

  ev = wv * xs * ea
  srep = jnp.dot(sexp, _head_expand_mat(), preferred_element_type=f32)
  u_ref[...] = ev * srep


def _tc_edge(esa, xs, xd, ea, cut, Wk1, Wk2, Wk3, Wv1, Wv2, Wv3, Wdot_r,
             interpret=False):
  B = EDGE_BLOCK
  grid = (N_EDGES // B,)
  def eb(j): return pl.BlockSpec((B, j), lambda i: (i, 0))
  def full(a): return pl.BlockSpec(a.shape, lambda i: (0,) * a.ndim)
  return pl.pallas_call(
      _edge_body,
      grid=grid,
      in_specs=[eb(N_RADIAL), eb(D), eb(D), eb(1), eb(1),
                full(Wk1), full(Wk2), full(Wk3),
                full(Wv1), full(Wv2), full(Wv3), full(Wdot_r)],
      out_specs=[eb(H), eb(D)],
      out_shape=[jax.ShapeDtypeStruct((N_EDGES, H), jnp.float32),
                 jax.ShapeDtypeStruct((N_EDGES, D), jnp.float32)],
      interpret=interpret,
  )(esa, xs, xd, ea, cut, Wk1, Wk2, Wk3, Wv1, Wv2, Wv3, Wdot_r)


def _final_body(acc_ref, zp_ref, wlin_ref, out_ref):
  nacc = acc_ref.shape[0]
  npart = zp_ref.shape[0]
  acc = acc_ref[0]
  for p in range(1, nacc):
    acc = acc + acc_ref[p]
  z = zp_ref[0]
  for p in range(1, npart):
    z = z + zp_ref[p]
  z = jnp.where(z == 0.0, 1.0, z)
  rs = lax.rsqrt(z)
  rsrep = jnp.dot(rs, _head_expand_mat(), preferred_element_type=jnp.float32)
  y = acc * rsrep
  out_ref[...] = jnp.dot(y, wlin_ref[...],
                         preferred_element_type=jnp.float32) * (1.0 / math.sqrt(D))


def _tc_final(acc, zpart, Wlin, interpret=False):
  B = NODE_BLOCK
  grid = (N_NODES // B,)
  na, np_ = acc.shape[0], zpart.shape[0]
  return pl.pallas_call(
      _final_body,
      grid=grid,
      in_specs=[pl.BlockSpec((na, B, D), lambda i: (0, i, 0)),
                pl.BlockSpec((np_, B, H), lambda i: (0, i, 0)),
                pl.BlockSpec((D, D), lambda i: (0, 0))],
      out_specs=pl.BlockSpec((B, D), lambda i: (i, 0)),
      out_shape=jax.ShapeDtypeStruct((N_NODES, D), jnp.float32),
      interpret=interpret,
  )(acc, zpart, Wlin)


def kernel(edge_src, edge_dst, edge_scalar_attr, edge_attr, edge_weight_cutoff,
           node_f, Wk1, Wk2, Wk3, Wv1, Wv2, Wv3, Wdot, Wlin):
  Wdot_r = jnp.transpose(Wdot, (1, 2, 0)).reshape(D, H * D)
  ea = edge_attr
  cut = edge_weight_cutoff[:, None]

  xs = node_f[edge_src]
  xd = node_f[edge_dst]

  expv, u = _tc_edge(edge_scalar_attr, xs, xd, ea, cut,
                     Wk1, Wk2, Wk3, Wv1, Wv2, Wv3, Wdot_r)

  zpart = jnp.zeros((N_NODES, H), jnp.float32).at[edge_dst].add(expv)[None]
  acc = jnp.zeros((N_NODES, D), jnp.float32).at[edge_dst].add(u)[None]

  return _tc_final(acc, zpart, Wlin)


# speedup vs baseline: 4.2515x; 2.6363x over previous
import functools
import math

import jax
import jax.numpy as jnp
from jax import lax
from jax.experimental import pallas as pl
from jax.experimental.pallas import tpu as pltpu
from jax.experimental.pallas import tpu_sc as plsc

N_NODES = 10000
N_EDGES = 320000
D = 128
H = 4
DH = D // H
N_RADIAL = 16
HIDDEN = 128

EDGE_BLOCK = 1000
NODE_BLOCK = 1000

NC = 2
NS = 16
NW = NC * NS
LANES = 16


def _head_expand_mat(dtype=jnp.float32):
  col = lax.broadcasted_iota(jnp.int32, (H, D), 1) // DH
  row = lax.broadcasted_iota(jnp.int32, (H, D), 0)
  return (col == row).astype(dtype)


def _edge_body(esa_ref, xs_ref, xd_ref, ea_ref, cut_ref,
               wk1_ref, wk2_ref, wk3_ref, wv1_ref, wv2_ref, wv3_ref,
               wdot_ref, exp_ref, u_ref):
  f32 = jnp.float32
  s_in = 1.0 / math.sqrt(N_RADIAL)
  s_h = 1.0 / math.sqrt(HIDDEN)

  esa = esa_ref[...]
  hk = jax.nn.gelu(jnp.dot(esa, wk1_ref[...], preferred_element_type=f32) * s_in)
  hk = jax.nn.gelu(jnp.dot(hk, wk2_ref[...], preferred_element_type=f32) * s_h)
  wk = jnp.dot(hk, wk3_ref[...], preferred_element_type=f32) * s_h
  hv = jax.nn.gelu(jnp.dot(esa, wv1_ref[...], preferred_element_type=f32) * s_in)
  hv = jax.nn.gelu(jnp.dot(hv, wv2_ref[...], preferred_element_type=f32) * s_h)
  wv = jnp.dot(hv, wv3_ref[...], preferred_element_type=f32) * s_h

  xs = xs_ref[...]
  ea = ea_ref[...]
  ek = wk * xs * ea
  m = jnp.dot(ek, wdot_ref[...], preferred_element_type=f32)
  xd = xd_ref[...]
  dots = []
  for w in range(H):
    dots.append(jnp.sum(m[:, w * D:(w + 1) * D] * xd, axis=1, keepdims=True))
  dot = jnp.concatenate(dots, axis=1) * (1.0 / D)

  cut = cut_ref[...]
  edot2 = jnp.exp(0.5 * dot)
  exp_ref[...] = cut * edot2 * edot2
  sexp = jnp.sqrt(cut) * edot2

  ev = wv * xs * ea
  srep = jnp.dot(sexp, _head_expand_mat(), preferred_element_type=f32)
  u_ref[...] = ev * srep


def _tc_edge(esa, xs, xd, ea, cut, Wk1, Wk2, Wk3, Wv1, Wv2, Wv3, Wdot_r,
             interpret=False):
  B = EDGE_BLOCK
  grid = (N_EDGES // B,)
  def eb(j): return pl.BlockSpec((B, j), lambda i: (i, 0))
  def full(a): return pl.BlockSpec(a.shape, lambda i: (0,) * a.ndim)
  return pl.pallas_call(
      _edge_body,
      grid=grid,
      in_specs=[eb(N_RADIAL), eb(D), eb(D), eb(1), eb(1),
                full(Wk1), full(Wk2), full(Wk3),
                full(Wv1), full(Wv2), full(Wv3), full(Wdot_r)],
      out_specs=[eb(H), eb(D)],
      out_shape=[jax.ShapeDtypeStruct((N_EDGES, H), jnp.float32),
                 jax.ShapeDtypeStruct((N_EDGES, D), jnp.float32)],
      interpret=interpret,
  )(esa, xs, xd, ea, cut, Wk1, Wk2, Wk3, Wv1, Wv2, Wv3, Wdot_r)


def _final_body(acc_ref, zp_ref, wlin_ref, out_ref):
  nacc = acc_ref.shape[0]
  npart = zp_ref.shape[0]
  acc = acc_ref[0]
  for p in range(1, nacc):
    acc = acc + acc_ref[p]
  z = zp_ref[0]
  for p in range(1, npart):
    z = z + zp_ref[p]
  z = jnp.where(z == 0.0, 1.0, z)
  rs = lax.rsqrt(z)
  rsrep = jnp.dot(rs, _head_expand_mat(), preferred_element_type=jnp.float32)
  y = acc * rsrep
  out_ref[...] = jnp.dot(y, wlin_ref[...],
                         preferred_element_type=jnp.float32) * (1.0 / math.sqrt(D))


def _tc_final(acc, zpart, Wlin, interpret=False):
  B = NODE_BLOCK
  grid = (N_NODES // B,)
  na, np_ = acc.shape[0], zpart.shape[0]
  return pl.pallas_call(
      _final_body,
      grid=grid,
      in_specs=[pl.BlockSpec((na, B, D), lambda i: (0, i, 0)),
                pl.BlockSpec((np_, B, H), lambda i: (0, i, 0)),
                pl.BlockSpec((D, D), lambda i: (0, 0))],
      out_specs=pl.BlockSpec((B, D), lambda i: (i, 0)),
      out_shape=jax.ShapeDtypeStruct((N_NODES, D), jnp.float32),
      interpret=interpret,
  )(acc, zpart, Wlin)


_EPW = N_EDGES // NW
_GK = 1000


def _sc_gather(node_f, edge_src, edge_dst):
  mesh = plsc.VectorSubcoreMesh(core_axis_name="c", subcore_axis_name="s")

  @functools.partial(
      pl.kernel,
      out_type=[jax.ShapeDtypeStruct((N_EDGES, D), jnp.float32),
                jax.ShapeDtypeStruct((N_EDGES, D), jnp.float32)],
      mesh=mesh,
      compiler_params=pltpu.CompilerParams(needs_layout_passes=False),
      scratch_types=[pltpu.VMEM((_GK,), jnp.int32),
                     pltpu.VMEM((_GK, D), jnp.float32),
                     pltpu.SemaphoreType.DMA],
  )
  def body(node_hbm, src_hbm, dst_hbm, xs_hbm, xd_hbm, idx_v, rows_v, sem):
    wid = lax.axis_index("s") * NC + lax.axis_index("c")

    def chunk(base, idx_hbm, out_hbm):
      pltpu.sync_copy(idx_hbm.at[pl.ds(base, _GK)], idx_v)
      pltpu.async_copy(node_hbm.at[idx_v], rows_v, sem).wait()
      pltpu.sync_copy(rows_v, out_hbm.at[pl.ds(base, _GK)])

    def loop_body(c, carry):
      base = wid * _EPW + c * _GK
      chunk(base, src_hbm, xs_hbm)
      chunk(base, dst_hbm, xd_hbm)
      return carry

    lax.fori_loop(0, _EPW // _GK, loop_body, 0)

  return body(node_f, edge_src, edge_dst)


def _sc_zscatter(expv_flat, edge_dst):
  mesh = plsc.VectorSubcoreMesh(core_axis_name="c", subcore_axis_name="s")
  ZW = N_NODES * H

  @functools.partial(
      pl.kernel,
      out_type=jax.ShapeDtypeStruct((NW * ZW,), jnp.float32),
      mesh=mesh,
      compiler_params=pltpu.CompilerParams(needs_layout_passes=False),
      scratch_types=[pltpu.VMEM((_EPW * H,), jnp.float32),
                     pltpu.VMEM((_EPW,), jnp.int32),
                     pltpu.VMEM((ZW,), jnp.float32)],
  )
  def body(exp_hbm, dst_hbm, zp_hbm, exp_v, dst_v, z_v):
    wid = lax.axis_index("s") * NC + lax.axis_index("c")
    pltpu.sync_copy(exp_hbm.at[pl.ds(wid * _EPW * H, _EPW * H)], exp_v)
    pltpu.sync_copy(dst_hbm.at[pl.ds(wid * _EPW, _EPW)], dst_v)

    zero = jnp.zeros((LANES,), jnp.float32)

    def zbody(i, carry):
      z_v[pl.ds(i * LANES, LANES)] = zero
      return carry

    lax.fori_loop(0, ZW // LANES, zbody, 0)

    lane = lax.iota(jnp.int32, LANES)
    lane_e = lane >> 2
    lane_h = lane & (H - 1)

    def sbody(g, carry):
      dstg = plsc.load_gather(dst_v, [g * 4 + lane_e])
      vals = exp_v[pl.ds(g * LANES, LANES)]
      plsc.addupdate_scatter(z_v, [dstg * H + lane_h], vals)
      return carry

    lax.fori_loop(0, _EPW * H // LANES, sbody, 0)
    pltpu.sync_copy(z_v, zp_hbm.at[pl.ds(wid * ZW, ZW)])

  return body(expv_flat, edge_dst)


_UK = 200
_DRAIN = 80
_NDCHUNK = N_NODES // _DRAIN


def _sc_uscatter(u, edge_dst, zero_nodes):
  mesh = plsc.VectorSubcoreMesh(core_axis_name="c", subcore_axis_name="s")

  @functools.partial(
      pl.kernel,
      out_type=jax.ShapeDtypeStruct((NC * N_NODES, D), jnp.float32),
      mesh=mesh,
      compiler_params=pltpu.CompilerParams(needs_layout_passes=False),
      scratch_types=[pltpu.VMEM((_UK, D), jnp.float32),
                     pltpu.VMEM((_UK,), jnp.int32),
                     pltpu.VMEM_SHARED((N_NODES, D), jnp.float32)],
  )
  def body(u_hbm, dst_hbm, zero_hbm, out_hbm, u_v, dst_v, acc_sh):
    cid = lax.axis_index("c")
    sid = lax.axis_index("s")
    wid = sid * NC + cid

    @pl.when(sid == 0)
    def _init():
      pltpu.sync_copy(zero_hbm, acc_sh)

    plsc.subcore_barrier()

    def cbody(k, carry):
      base = wid * _EPW + k * _UK
      pltpu.sync_copy(dst_hbm.at[pl.ds(base, _UK)], dst_v)
      pltpu.sync_copy(u_hbm.at[pl.ds(base, _UK)], u_v)
      pltpu.sync_copy(u_v, acc_sh.at[dst_v], add=True)
      return carry

    lax.fori_loop(0, _EPW // _UK, cbody, 0)
    plsc.subcore_barrier()

    def dbody(j, carry):
      c = sid + j * NS

      @pl.when(c < _NDCHUNK)
      def _():
        row = c * _DRAIN
        dr_v = u_v.at[pl.ds(0, _DRAIN)]
        pltpu.sync_copy(acc_sh.at[pl.ds(row, _DRAIN)], dr_v)
        pltpu.sync_copy(dr_v, out_hbm.at[pl.ds(cid * N_NODES + row, _DRAIN)])

      return carry

    lax.fori_loop(0, (_NDCHUNK + NS - 1) // NS, dbody, 0)

  return body(u, edge_dst, zero_nodes)


def kernel(edge_src, edge_dst, edge_scalar_attr, edge_attr, edge_weight_cutoff,
           node_f, Wk1, Wk2, Wk3, Wv1, Wv2, Wv3, Wdot, Wlin):
  Wdot_r = jnp.transpose(Wdot, (1, 2, 0)).reshape(D, H * D)
  ea = edge_attr
  cut = edge_weight_cutoff[:, None]

  xs, xd = _sc_gather(node_f, edge_src, edge_dst)

  expv, u = _tc_edge(edge_scalar_attr, xs, xd, ea, cut,
                     Wk1, Wk2, Wk3, Wv1, Wv2, Wv3, Wdot_r)

  zpart = _sc_zscatter(expv.reshape(-1), edge_dst).reshape(NW, N_NODES, H)
  zero_nodes = jnp.zeros((N_NODES, D), jnp.float32)
  acc = _sc_uscatter(u, edge_dst, zero_nodes).reshape(NC, N_NODES, D)

  return _tc_final(acc, zpart, Wlin)
